# Initial kernel scaffold; baseline (speedup 1.0000x reference)
import jax
import jax.numpy as jnp
from jax.experimental import pallas as pl


def kernel(x, edge_index, params):
    raise NotImplementedError("write your pallas kernel here")


# speedup vs baseline: 4.5997x; 4.5997x over previous
import functools

import jax
import jax.numpy as jnp
from jax import lax
from jax.experimental import pallas as pl
from jax.experimental.pallas import tpu as pltpu
from jax.experimental.pallas import tpu_sc as plsc

N_SEG = 10000
NNZ = 320000
D = 128
NCLS = 40

NC, NS = 2, 16
NW = NC * NS
E_PER_TILE = NNZ // NW
CHUNK = 80
N_CHUNKS = E_PER_TILE // CHUNK
N_PAD = 10240
ROWS_PER_TILE = N_PAD // NS


def _sc_propagate_body(h_hbm, src_hbm, dst_hbm, z2_hbm, z1_hbm,
                       sums_hbm, cnts_hbm,
                       sidx_v, didx_v, rows_v, ones_v, acc_sh, cnt_sh, sem):
    c = lax.axis_index("c")
    s = lax.axis_index("s")
    wid = s * NC + c

    pltpu.sync_copy(z2_hbm.at[pl.ds(s * ROWS_PER_TILE, ROWS_PER_TILE)],
                    acc_sh.at[pl.ds(s * ROWS_PER_TILE, ROWS_PER_TILE)])

    @pl.when(s == 0)
    def _():
        pltpu.sync_copy(z1_hbm, cnt_sh)

    for i in range(CHUNK // 16):
        ones_v[pl.ds(i * 16, 16)] = jnp.full((16,), 1.0, jnp.float32)

    plsc.subcore_barrier()

    base0 = wid * E_PER_TILE

    def step(i, carry):
        base = base0 + i * CHUNK
        pltpu.sync_copy(src_hbm.at[pl.ds(base, CHUNK)], sidx_v)
        pltpu.sync_copy(dst_hbm.at[pl.ds(base, CHUNK)], didx_v)
        pltpu.async_copy(h_hbm.at[sidx_v], rows_v, sem).wait()
        pltpu.sync_copy(rows_v, acc_sh.at[didx_v], add=True)
        pltpu.sync_copy(ones_v, cnt_sh.at[didx_v], add=True)
        return carry

    lax.fori_loop(0, N_CHUNKS, step, 0)

    plsc.subcore_barrier()

    pltpu.sync_copy(acc_sh.at[pl.ds(s * ROWS_PER_TILE, ROWS_PER_TILE)],
                    sums_hbm.at[c, pl.ds(s * ROWS_PER_TILE, ROWS_PER_TILE)])

    @pl.when(s == 0)
    def _():
        pltpu.sync_copy(cnt_sh, cnts_hbm.at[c])


_sc_propagate = pl.kernel(
    _sc_propagate_body,
    out_type=(jax.ShapeDtypeStruct((NC, N_PAD, D), jnp.float32),
              jax.ShapeDtypeStruct((NC, N_PAD), jnp.float32)),
    mesh=plsc.VectorSubcoreMesh(core_axis_name="c", subcore_axis_name="s"),
    scratch_types=[
        pltpu.VMEM((CHUNK,), jnp.int32),
        pltpu.VMEM((CHUNK,), jnp.int32),
        pltpu.VMEM((CHUNK, D), jnp.float32),
        pltpu.VMEM((CHUNK,), jnp.float32),
        pltpu.VMEM_SHARED((N_PAD, D), jnp.float32),
        pltpu.VMEM_SHARED((N_PAD,), jnp.float32),
        pltpu.SemaphoreType.DMA,
    ],
)


R = 2000
GRID = N_SEG // R

_HI = jax.lax.Precision.HIGHEST


def _dot(a, b):
    return jax.lax.dot_general(a, b, (((1,), (0,)), ((), ())),
                               precision=_HI,
                               preferred_element_type=jnp.float32)


def _enc_body(x_ref, w1, b1, w2, b2, o_ref):
    t = jnp.maximum(_dot(x_ref[...], w1[...]) + b1[...], 0.0)
    o_ref[...] = jnp.maximum(_dot(t, w2[...]) + b2[...], 0.0)


def _mid_body(s_ref, c_ref, wd1, bd1, wd2, bd2, we1, be1, we2, be2, o_ref):
    cb = c_ref[...]
    cnt = cb[:, 0:1] + cb[:, 1:2]
    inv = 1.0 / jnp.maximum(cnt, 1.0)
    agg = (s_ref[0] + s_ref[1]) * inv
    t = jnp.maximum(_dot(agg, wd1[...]) + bd1[...], 0.0)
    t = jnp.maximum(_dot(t, wd2[...]) + bd2[...], 0.0)
    t = jnp.maximum(_dot(t, we1[...]) + be1[...], 0.0)
    o_ref[...] = jnp.maximum(_dot(t, we2[...]) + be2[...], 0.0)


def _fin_body(s_ref, c_ref, wd1, bd1, wd2, bd2, wc1, bc1, wc2, bc2, o_ref):
    cb = c_ref[...]
    cnt = cb[:, 0:1] + cb[:, 1:2]
    inv = 1.0 / jnp.maximum(cnt, 1.0)
    agg = (s_ref[0] + s_ref[1]) * inv
    t = jnp.maximum(_dot(agg, wd1[...]) + bd1[...], 0.0)
    t = jnp.maximum(_dot(t, wd2[...]) + bd2[...], 0.0)
    t = jnp.maximum(_dot(t, wc1[...]) + bc1[...], 0.0)
    o_ref[...] = _dot(t, wc2[...]) + bc2[...]


def _wspec(shape):
    return pl.BlockSpec(shape, lambda i: (0,) * len(shape))


def _make_enc():
    return pl.pallas_call(
        _enc_body,
        grid=(GRID,),
        in_specs=[pl.BlockSpec((R, D), lambda i: (i, 0)),
                  _wspec((D, D)), _wspec((1, D)), _wspec((D, D)), _wspec((1, D))],
        out_specs=pl.BlockSpec((R, D), lambda i: (i, 0)),
        out_shape=jax.ShapeDtypeStruct((N_SEG, D), jnp.float32),
    )


def _make_mid(body, out_cols):
    n_w = 4
    wspecs = []
    for _ in range(n_w - 1):
        wspecs += [_wspec((D, D)), _wspec((1, D))]
    wspecs += [_wspec((D, out_cols)), _wspec((1, out_cols))]
    return pl.pallas_call(
        body,
        grid=(GRID,),
        in_specs=[pl.BlockSpec((NC, R, D), lambda i: (0, i, 0)),
                  pl.BlockSpec((R, NC), lambda i: (i, 0))] + wspecs,
        out_specs=pl.BlockSpec((R, out_cols), lambda i: (i, 0)),
        out_shape=jax.ShapeDtypeStruct((N_SEG, out_cols), jnp.float32),
    )


_enc_call = _make_enc()
_mid_call = _make_mid(_mid_body, D)
_fin_call = _make_mid(_fin_body, NCLS)


def _unpack(layers):
    (w1, b1), (w2, b2) = layers
    return w1, b1.reshape(1, -1), w2, b2.reshape(1, -1)


def kernel(x, edge_index, params):
    src = edge_index[0]
    dst = edge_index[1]
    z2 = jnp.zeros((N_PAD, D), jnp.float32)
    z1 = jnp.zeros((N_PAD,), jnp.float32)

    g = _enc_call(x, *_unpack(params["V2E"][0]["enc"]))

    s0, c0 = _sc_propagate(g, src, dst, z2, z1)
    g = _mid_call(s0, c0.T, *_unpack(params["V2E"][0]["dec"]),
                  *_unpack(params["E2V"][0]["enc"]))

    s1, c1 = _sc_propagate(g, dst, src, z2, z1)
    g = _mid_call(s1, c1.T, *_unpack(params["E2V"][0]["dec"]),
                  *_unpack(params["V2E"][1]["enc"]))

    s2, c2 = _sc_propagate(g, src, dst, z2, z1)
    g = _mid_call(s2, c2.T, *_unpack(params["V2E"][1]["dec"]),
                  *_unpack(params["E2V"][1]["enc"]))

    s3, c3 = _sc_propagate(g, dst, src, z2, z1)
    out = _fin_call(s3, c3.T, *_unpack(params["E2V"][1]["dec"]),
                    *_unpack(params["clf"]))
    return out


# speedup vs baseline: 4.6222x; 1.0049x over previous
import jax
import jax.numpy as jnp
from jax import lax
from jax.experimental import pallas as pl
from jax.experimental.pallas import tpu as pltpu
from jax.experimental.pallas import tpu_sc as plsc

N_SEG = 10000
NNZ = 320000
D = 128
DH = D // 2
NCLS = 40

NC, NS = 2, 16
CHUNK = 128
NNZ_PAD = 327680
E_PER_TILE = NNZ_PAD // NS
N_CHUNKS = E_PER_TILE // CHUNK
DUMMY_ROW = N_SEG
N_PAD = 10112
ROWS_PER_TILE = N_PAD // NS


def _sc_propagate_body(h_hbm, src_hbm, dst_hbm, z2_hbm, z1_hbm,
                       sums_hbm, cnt_hbm,
                       sidx_v, didx_v, rows_v, ones_v, acc_sh, cnt_sh,
                       gs0, gs1, ss0, ss1, csem):
    c = lax.axis_index("c")
    s = lax.axis_index("s")

    pltpu.sync_copy(z2_hbm.at[pl.ds(s * ROWS_PER_TILE, ROWS_PER_TILE)],
                    acc_sh.at[pl.ds(s * ROWS_PER_TILE, ROWS_PER_TILE)])

    @pl.when(jnp.logical_and(s == 0, c == 0))
    def _():
        pltpu.sync_copy(z1_hbm, cnt_sh)

    for i in range(CHUNK // 16):
        ones_v[pl.ds(i * 16, 16)] = jnp.full((16,), 1.0, jnp.float32)

    pltpu.sync_copy(src_hbm.at[s], sidx_v)
    pltpu.sync_copy(dst_hbm.at[s], didx_v)

    plsc.subcore_barrier()

    def _gather(i, buf, sem):
        return pltpu.async_copy(h_hbm.at[c].at[sidx_v.at[i]], rows_v.at[buf],
                                sem)

    def _scatter(i, buf, sem):
        d = pltpu.async_copy(rows_v.at[buf], acc_sh.at[didx_v.at[i]], sem,
                             add=True)

        @pl.when(c == 0)
        def _():
            pltpu.async_copy(ones_v, cnt_sh.at[didx_v.at[i]], csem, add=True)
        return d

    _gather(0, 0, gs0)

    def step(g, carry):
        i0 = 2 * g
        i1 = i0 + 1

        @pl.when(g > 0)
        def _():
            pltpu.make_async_copy(rows_v.at[1], acc_sh.at[didx_v.at[0]],
                                  ss1).wait()
        _gather(i1, 1, gs1)
        pltpu.make_async_copy(h_hbm.at[c].at[sidx_v.at[0]], rows_v.at[0],
                              gs0).wait()
        _scatter(i0, 0, ss0).wait()

        @pl.when(i0 + 2 < N_CHUNKS)
        def _():
            _gather(i0 + 2, 0, gs0)
        pltpu.make_async_copy(h_hbm.at[c].at[sidx_v.at[0]], rows_v.at[1],
                              gs1).wait()
        _scatter(i1, 1, ss1)
        return carry

    lax.fori_loop(0, N_CHUNKS // 2, step, 0)

    pltpu.make_async_copy(rows_v.at[1], acc_sh.at[didx_v.at[0]], ss1).wait()

    @pl.when(c == 0)
    def _():
        pltpu.make_async_copy(src_hbm.at[s], sidx_v, csem).wait()

    plsc.subcore_barrier()

    pltpu.sync_copy(acc_sh.at[pl.ds(s * ROWS_PER_TILE, ROWS_PER_TILE)],
                    sums_hbm.at[c, pl.ds(s * ROWS_PER_TILE, ROWS_PER_TILE)])

    @pl.when(jnp.logical_and(s == 0, c == 0))
    def _():
        pltpu.sync_copy(cnt_sh, cnt_hbm)


_sc_propagate = pl.kernel(
    _sc_propagate_body,
    out_type=(jax.ShapeDtypeStruct((NC, N_PAD, DH), jnp.float32),
              jax.ShapeDtypeStruct((N_PAD,), jnp.float32)),
    mesh=plsc.VectorSubcoreMesh(core_axis_name="c", subcore_axis_name="s"),
    compiler_params=pltpu.CompilerParams(use_tc_tiling_on_sc=False),
    scratch_types=[
        pltpu.VMEM((N_CHUNKS, CHUNK), jnp.int32),
        pltpu.VMEM((N_CHUNKS, CHUNK), jnp.int32),
        pltpu.VMEM((2, CHUNK, DH), jnp.float32),
        pltpu.VMEM((CHUNK,), jnp.float32),
        pltpu.VMEM_SHARED((N_PAD, DH), jnp.float32),
        pltpu.VMEM_SHARED((N_PAD,), jnp.float32),
        pltpu.SemaphoreType.DMA,
        pltpu.SemaphoreType.DMA,
        pltpu.SemaphoreType.DMA,
        pltpu.SemaphoreType.DMA,
        pltpu.SemaphoreType.DMA,
    ],
)


R = 2000
GRID = N_SEG // R

_HI = jax.lax.Precision.HIGHEST


def _dot(a, b):
    return jax.lax.dot_general(a, b, (((1,), (0,)), ((), ())),
                               precision=_HI,
                               preferred_element_type=jnp.float32)


def _split_store(o_ref, g):
    o_ref[0, :, :] = g[:, :DH]
    o_ref[1, :, :] = g[:, DH:]


def _enc_body(x_ref, w1, b1, w2, b2, o_ref):
    t = jnp.maximum(_dot(x_ref[...], w1[...]) + b1[...], 0.0)
    _split_store(o_ref, jnp.maximum(_dot(t, w2[...]) + b2[...], 0.0))


def _agg(s_ref, c_ref):
    inv = 1.0 / jnp.maximum(c_ref[...], 1.0)
    return jnp.concatenate((s_ref[0], s_ref[1]), axis=1) * inv


def _mid_body(s_ref, c_ref, wd1, bd1, wd2, bd2, we1, be1, we2, be2, o_ref):
    t = _agg(s_ref, c_ref)
    t = jnp.maximum(_dot(t, wd1[...]) + bd1[...], 0.0)
    t = jnp.maximum(_dot(t, wd2[...]) + bd2[...], 0.0)
    t = jnp.maximum(_dot(t, we1[...]) + be1[...], 0.0)
    _split_store(o_ref, jnp.maximum(_dot(t, we2[...]) + be2[...], 0.0))


def _fin_body(s_ref, c_ref, wd1, bd1, wd2, bd2, wc1, bc1, wc2, bc2, o_ref):
    t = _agg(s_ref, c_ref)
    t = jnp.maximum(_dot(t, wd1[...]) + bd1[...], 0.0)
    t = jnp.maximum(_dot(t, wd2[...]) + bd2[...], 0.0)
    t = jnp.maximum(_dot(t, wc1[...]) + bc1[...], 0.0)
    o_ref[...] = _dot(t, wc2[...]) + bc2[...]


def _wspec(shape):
    return pl.BlockSpec(shape, lambda i: (0,) * len(shape))


_SPLIT_OUT = pl.BlockSpec((NC, R, DH), lambda i: (0, i, 0))
_SPLIT_SHAPE = jax.ShapeDtypeStruct((NC, N_PAD, DH), jnp.float32)


def _make_enc():
    return pl.pallas_call(
        _enc_body,
        grid=(GRID,),
        in_specs=[pl.BlockSpec((R, D), lambda i: (i, 0)),
                  _wspec((D, D)), _wspec((1, D)), _wspec((D, D)), _wspec((1, D))],
        out_specs=_SPLIT_OUT,
        out_shape=_SPLIT_SHAPE,
    )


def _make_mid(body, final):
    wspecs = []
    for _ in range(3):
        wspecs += [_wspec((D, D)), _wspec((1, D))]
    out_cols = NCLS if final else D
    wspecs += [_wspec((D, out_cols)), _wspec((1, out_cols))]
    return pl.pallas_call(
        body,
        grid=(GRID,),
        in_specs=[pl.BlockSpec((NC, R, DH), lambda i: (0, i, 0)),
                  pl.BlockSpec((R, 1), lambda i: (i, 0))] + wspecs,
        out_specs=(pl.BlockSpec((R, NCLS), lambda i: (i, 0)) if final
                   else _SPLIT_OUT),
        out_shape=(jax.ShapeDtypeStruct((N_SEG, NCLS), jnp.float32) if final
                   else _SPLIT_SHAPE),
    )


_enc_call = _make_enc()
_mid_call = _make_mid(_mid_body, final=False)
_fin_call = _make_mid(_fin_body, final=True)


def _unpack(layers):
    (w1, b1), (w2, b2) = layers
    return w1, b1.reshape(1, -1), w2, b2.reshape(1, -1)


def kernel(x, edge_index, params):
    pad = NNZ_PAD - NNZ
    src = jnp.concatenate(
        [edge_index[0], jnp.full((pad,), DUMMY_ROW, jnp.int32)]).reshape(
            NS, N_CHUNKS, CHUNK)
    dst = jnp.concatenate(
        [edge_index[1], jnp.full((pad,), DUMMY_ROW, jnp.int32)]).reshape(
            NS, N_CHUNKS, CHUNK)
    z2 = jnp.zeros((N_PAD, DH), jnp.float32)
    z1 = jnp.zeros((N_PAD,), jnp.float32)

    g = _enc_call(x, *_unpack(params["V2E"][0]["enc"]))

    s0, c0 = _sc_propagate(g, src, dst, z2, z1)
    g = _mid_call(s0, c0.reshape(-1, 1), *_unpack(params["V2E"][0]["dec"]),
                  *_unpack(params["E2V"][0]["enc"]))

    s1, c1 = _sc_propagate(g, dst, src, z2, z1)
    g = _mid_call(s1, c1.reshape(-1, 1), *_unpack(params["E2V"][0]["dec"]),
                  *_unpack(params["V2E"][1]["enc"]))

    s2, c2 = _sc_propagate(g, src, dst, z2, z1)
    g = _mid_call(s2, c2.reshape(-1, 1), *_unpack(params["V2E"][1]["dec"]),
                  *_unpack(params["E2V"][1]["enc"]))

    s3, c3 = _sc_propagate(g, dst, src, z2, z1)
    out = _fin_call(s3, c3.reshape(-1, 1), *_unpack(params["E2V"][1]["dec"]),
                    *_unpack(params["clf"]))
    return out
